# baseline (speedup 1.0000x reference)
import functools

import jax
import jax.numpy as jnp
import numpy as np
from jax import lax
from jax.experimental import pallas as pl
from jax.experimental.pallas import tpu as pltpu
from jax.experimental.pallas import tpu_sc as plsc

_B = 16384
_NC, _NS = 2, 16
_NW = _NC * _NS
_BPW = _B // _NW
_D = 8
_LOG2PI = float(np.log(2.0 * np.pi))


def _sc_gather_body(tool_hbm, tab_hbm, g_out, idx_v, g_v, sem):
    wid = lax.axis_index("s") * _NC + lax.axis_index("c")
    base = wid * _BPW
    pltpu.sync_copy(tool_hbm.at[pl.ds(base, _BPW)], idx_v)
    pltpu.async_copy(tab_hbm.at[idx_v], g_v, sem).wait()
    pltpu.sync_copy(g_v, g_out.at[pl.ds(base, _BPW)])


@functools.cache
def _sc_gather():
    return pl.kernel(
        _sc_gather_body,
        out_type=jax.ShapeDtypeStruct((_B, _D), jnp.float32),
        mesh=plsc.VectorSubcoreMesh(core_axis_name="c", subcore_axis_name="s",
                                    num_cores=_NC, num_subcores=_NS),
        scratch_types=[
            pltpu.VMEM((_BPW,), jnp.int32),
            pltpu.VMEM((_BPW, _D), jnp.float32),
            pltpu.SemaphoreType.DMA,
        ],
        compiler_params=pltpu.CompilerParams(use_tc_tiling_on_sc=False),
    )


def _tc_combine_body(t_ref, px_ref, py_ref, mx_ref, my_ref, lx_ref, ly_ref,
                     tg_ref, o_ref):
    t = t_ref[...]
    m = jnp.max(t)
    s = jnp.sum(jnp.exp(t - m))
    log_z = m + jnp.log(s)
    px = px_ref[...]
    py = py_ref[...]
    mx = mx_ref[...]
    my = my_ref[...]
    lx = lx_ref[...]
    ly = ly_ref[...]
    q = (px - mx) ** 2 * jnp.exp(-lx) + (py - my) ** 2 * jnp.exp(-ly)
    o_ref[...] = tg_ref[...] - log_z - 0.5 * q - 0.5 * (lx + ly) - _LOG2PI


def kernel(action, tool_distribution, log_std, means):
    n = tool_distribution.shape[0]
    tool = action[:, 0].astype(jnp.int32)
    fused = jnp.concatenate(
        [tool_distribution[:, None], means, log_std,
         jnp.zeros((n, _D - 5), jnp.float32)], axis=1)
    g = _sc_gather()(tool, fused)

    pad = (-n) % 128
    tpad = jnp.pad(tool_distribution, (0, pad), constant_values=-jnp.inf)
    t2d = tpad.reshape(-1, 128)
    r = lambda x: x.reshape(128, 128)
    out = pl.pallas_call(
        _tc_combine_body,
        out_shape=jax.ShapeDtypeStruct((128, 128), jnp.float32),
    )(t2d, r(action[:, 1]), r(action[:, 2]), r(g[:, 1]), r(g[:, 2]),
      r(g[:, 3]), r(g[:, 4]), r(g[:, 0]))
    return out.reshape(_B)
